# baseline (speedup 1.0000x reference)
import functools

import jax
import jax.numpy as jnp
from jax import lax
from jax.experimental import pallas as pl
from jax.experimental.pallas import tpu as pltpu, tpu_sc as plsc

NC = 2
NS = 16
NW = NC * NS
L = 16

D = 2048
ROWS = 2 * 4096
ROWS_PER_W = ROWS // NW
R = 8
CHUNKS = ROWS_PER_W // R


def _permute_body(x_hbm, perm_hbm, out_hbm, perm_v, in_buf, out_buf, sem):
    wid = lax.axis_index("s") * NC + lax.axis_index("c")
    base_row = wid * ROWS_PER_W

    pltpu.sync_copy(perm_hbm, perm_v)

    @pl.loop(0, CHUNKS)
    def _chunk(i):
        e0 = (base_row + i * R) * D
        pltpu.sync_copy(x_hbm.at[pl.ds(e0, R * D)], in_buf)

        @pl.loop(0, D // L)
        def _col(c):
            off = c * L
            idx = perm_v[pl.ds(off, L)]
            for r in range(R):
                out_buf[pl.ds(r * D + off, L)] = plsc.load_gather(
                    in_buf, [idx + (r * D)]
                )

        pltpu.sync_copy(out_buf, out_hbm.at[pl.ds(e0, R * D)])


@jax.jit
def _permute(x2d, perm32):
    mesh = plsc.VectorSubcoreMesh(core_axis_name="c", subcore_axis_name="s")
    return pl.kernel(
        _permute_body,
        out_type=jax.ShapeDtypeStruct((ROWS * D,), jnp.float32),
        mesh=mesh,
        compiler_params=pltpu.CompilerParams(
            needs_layout_passes=False,
            use_tc_tiling_on_sc=False,
        ),
        scratch_types=[
            pltpu.VMEM((D,), jnp.int32),
            pltpu.VMEM((R * D,), jnp.float32),
            pltpu.VMEM((R * D,), jnp.float32),
            pltpu.SemaphoreType.DMA,
        ],
    )(x2d, perm32)


def kernel(x, perm):
    out = _permute(x.reshape(ROWS * D), perm.astype(jnp.int32))
    return out.reshape(x.shape)


# speedup vs baseline: 1.2194x; 1.2194x over previous
import jax
import jax.numpy as jnp
from jax import lax
from jax.experimental import pallas as pl
from jax.experimental.pallas import tpu as pltpu, tpu_sc as plsc

NC = 2
NS = 16
NW = NC * NS
L = 16

D = 2048
ROWS = 2 * 4096
ROWS_PER_W = ROWS // NW
R = 8
CHUNKS = ROWS_PER_W // R


def _permute_body(
    x_hbm, perm_hbm, out_hbm,
    perm_v, in0, in1, o0, o1, s_in0, s_in1, s_out0, s_out1,
):
    wid = lax.axis_index("s") * NC + lax.axis_index("c")
    base = wid * (ROWS_PER_W * D)

    def in_slice(i):
        return x_hbm.at[pl.ds(base + i * (R * D), R * D)]

    def out_slice(i):
        return out_hbm.at[pl.ds(base + i * (R * D), R * D)]

    def compute(ib, ob):
        @pl.loop(0, D // L)
        def _col(c):
            off = c * L
            idx = perm_v[pl.ds(off, L)]
            for r in range(R):
                ob[pl.ds(r * D + off, L)] = plsc.load_gather(
                    ib, [idx + (r * D)]
                )

    pltpu.sync_copy(perm_hbm, perm_v)
    pltpu.async_copy(in_slice(0), in0, s_in0)
    pltpu.async_copy(in_slice(1), in1, s_in1)

    @pl.loop(0, CHUNKS, step=2)
    def _outer(i):
        for b, (ib, ob, si, so) in enumerate(
            ((in0, o0, s_in0, s_out0), (in1, o1, s_in1, s_out1))
        ):
            ci = i + b
            pltpu.make_async_copy(in_slice(ci), ib, si).wait()

            @pl.when(ci >= 2)
            def _drain_out():
                pltpu.make_async_copy(ob, out_slice(ci - 2), so).wait()

            compute(ib, ob)
            pltpu.async_copy(ob, out_slice(ci), so)

            @pl.when(ci + 2 < CHUNKS)
            def _prefetch_in():
                pltpu.async_copy(in_slice(ci + 2), ib, si)

    pltpu.make_async_copy(o0, out_slice(CHUNKS - 2), s_out0).wait()
    pltpu.make_async_copy(o1, out_slice(CHUNKS - 1), s_out1).wait()


@jax.jit
def _permute(x_flat, perm32):
    mesh = plsc.VectorSubcoreMesh(core_axis_name="c", subcore_axis_name="s")
    return pl.kernel(
        _permute_body,
        out_type=jax.ShapeDtypeStruct((ROWS * D,), jnp.float32),
        mesh=mesh,
        compiler_params=pltpu.CompilerParams(
            needs_layout_passes=False,
            use_tc_tiling_on_sc=False,
        ),
        scratch_types=[
            pltpu.VMEM((D,), jnp.int32),
            pltpu.VMEM((R * D,), jnp.float32),
            pltpu.VMEM((R * D,), jnp.float32),
            pltpu.VMEM((R * D,), jnp.float32),
            pltpu.VMEM((R * D,), jnp.float32),
            pltpu.SemaphoreType.DMA,
            pltpu.SemaphoreType.DMA,
            pltpu.SemaphoreType.DMA,
            pltpu.SemaphoreType.DMA,
        ],
    )(x_flat, perm32)


def kernel(x, perm):
    out = _permute(x.reshape(ROWS * D), perm.astype(jnp.int32))
    return out.reshape(x.shape)


# speedup vs baseline: 1.8438x; 1.5121x over previous
import jax
import jax.numpy as jnp
from jax import lax
from jax.experimental import pallas as pl
from jax.experimental.pallas import tpu as pltpu, tpu_sc as plsc

NC = 2
NS = 16
NW = NC * NS
L = 16

D = 2048
ROWS = 2 * 4096
ROWS_PER_W = ROWS // NW
R = 8
CHUNKS = ROWS_PER_W // R


def _permute_body(
    x_hbm, perm_hbm, out_hbm,
    perm_v, in0, in1, o0, o1, s_in0, s_in1, s_out0, s_out1,
):
    wid = lax.axis_index("s") * NC + lax.axis_index("c")
    base = wid * (ROWS_PER_W * D)

    def in_slice(i):
        return x_hbm.at[pl.ds(base + i * (R * D), R * D)]

    def out_slice(i):
        return out_hbm.at[pl.ds(base + i * (R * D), R * D)]

    def compute(ib, ob):
        @plsc.parallel_loop(0, D, step=L, unroll=4)
        def _col(off):
            idx = perm_v[pl.ds(off, L)]
            for r in range(R):
                ob[pl.ds(r * D + off, L)] = plsc.load_gather(
                    ib, [idx + (r * D)]
                )

    pltpu.sync_copy(perm_hbm, perm_v)
    pltpu.async_copy(in_slice(0), in0, s_in0)
    pltpu.async_copy(in_slice(1), in1, s_in1)

    @pl.loop(0, CHUNKS, step=2)
    def _outer(i):
        for b, (ib, ob, si, so) in enumerate(
            ((in0, o0, s_in0, s_out0), (in1, o1, s_in1, s_out1))
        ):
            ci = i + b
            pltpu.make_async_copy(in_slice(ci), ib, si).wait()

            @pl.when(ci >= 2)
            def _drain_out():
                pltpu.make_async_copy(ob, out_slice(ci - 2), so).wait()

            compute(ib, ob)
            pltpu.async_copy(ob, out_slice(ci), so)

            @pl.when(ci + 2 < CHUNKS)
            def _prefetch_in():
                pltpu.async_copy(in_slice(ci + 2), ib, si)

    pltpu.make_async_copy(o0, out_slice(CHUNKS - 2), s_out0).wait()
    pltpu.make_async_copy(o1, out_slice(CHUNKS - 1), s_out1).wait()


@jax.jit
def _permute(x_flat, perm32):
    mesh = plsc.VectorSubcoreMesh(core_axis_name="c", subcore_axis_name="s")
    return pl.kernel(
        _permute_body,
        out_type=jax.ShapeDtypeStruct((ROWS * D,), jnp.float32),
        mesh=mesh,
        compiler_params=pltpu.CompilerParams(
            needs_layout_passes=False,
            use_tc_tiling_on_sc=False,
        ),
        scratch_types=[
            pltpu.VMEM((D,), jnp.int32),
            pltpu.VMEM((R * D,), jnp.float32),
            pltpu.VMEM((R * D,), jnp.float32),
            pltpu.VMEM((R * D,), jnp.float32),
            pltpu.VMEM((R * D,), jnp.float32),
            pltpu.SemaphoreType.DMA,
            pltpu.SemaphoreType.DMA,
            pltpu.SemaphoreType.DMA,
            pltpu.SemaphoreType.DMA,
        ],
    )(x_flat, perm32)


def kernel(x, perm):
    out = _permute(x.reshape(ROWS * D), perm.astype(jnp.int32))
    return out.reshape(x.shape)
